# Initial kernel scaffold; baseline (speedup 1.0000x reference)
import jax
import jax.numpy as jnp
from jax.experimental import pallas as pl


def kernel(recipe_x, usr_rcp_edges, rcp_usr_edges, usr_rcp_weights, rcp_usr_weights, usr_emb, rcp_emb):
    raise NotImplementedError("write your pallas kernel here")


# speedup vs baseline: 5.1012x; 5.1012x over previous
import functools

import jax
import jax.numpy as jnp
from jax import lax
from jax.experimental import pallas as pl
from jax.experimental.pallas import tpu as pltpu
from jax.experimental.pallas import tpu_sc as plsc

N = 50000
NP = 50048
D = 64
C = 32
E = 800000
NC, NS, L = 2, 16, 16

EROWS = 6272
EPAD = EROWS * 128 - E
RPT = EROWS // NS
KB = 4
NB = RPT // KB
RPT_N = NP // NS
RB = 184
NRB = RPT_N // RB
ALPHA = 1.0 / 4.0

def _bcast(w16, e):
    return jnp.take_along_axis(w16, jnp.full((16,), e, jnp.int32), axis=0)


def _body(ux0, rx0, se_ur, de_ur, w_ur, se_ru, de_ru, w_ru,
          uout, rout, xu, xr,
          acc, rows, sidx, didx, wv):
    cid = lax.axis_index("c")
    sid = lax.axis_index("s")
    rbase = sid * RPT_N
    ebase = sid * RPT

    TA, TB = 0, RB

    def zrow(r, carry):
        z16 = lax.broadcast(jnp.float32(0.0), (16,))
        rows[r, 0:16] = z16
        rows[r, 16:32] = z16
        return carry
    lax.fori_loop(0, RB, zrow, 0, unroll=8)

    def zacc(b, carry):
        off = pl.multiple_of(rbase + b * RB, 8)
        pltpu.sync_copy(rows.at[pl.ds(TA, RB)], acc.at[pl.ds(off, RB)])
        return carry
    lax.fori_loop(0, NRB, zacc, 0)
    plsc.subcore_barrier()

    def scatter_phase(xsrc, se, de, wh):
        def batch(b, carry):
            r0 = ebase + b * KB
            pltpu.sync_copy(se.at[pl.ds(r0, KB)], sidx)
            pltpu.sync_copy(de.at[pl.ds(r0, KB)], didx)
            pltpu.sync_copy(wh.at[pl.ds(r0, KB)], wv)
            for j in range(KB):
                pltpu.sync_copy(xsrc.at[sidx.at[j]],
                                rows.at[pl.ds(j * 128, 128)])
            def mgrp(g, carry2):
                w16 = wv[lax.div(g, 8), pl.ds(lax.rem(g, 8) * 16, 16)]
                base = g * 16

                def medge(e, carry3):
                    wb = _bcast(w16, e)
                    rows[base + e, 0:16] = rows[base + e, 0:16] * wb
                    rows[base + e, 16:32] = rows[base + e, 16:32] * wb
                    return carry3
                lax.fori_loop(0, 16, medge, 0, unroll=4)
                return carry2
            lax.fori_loop(0, KB * 8, mgrp, 0)
            for j in range(KB):
                pltpu.sync_copy(rows.at[pl.ds(j * 128, 128)],
                                acc.at[didx.at[j]], add=True)
            return carry
        lax.fori_loop(0, NB, batch, 0)

    def drain(src_ref, out_ref, xdst, first):
        def dblk(b, carry):
            off = pl.multiple_of(rbase + b * RB, 8)
            sl = pl.ds(off, RB)
            pltpu.sync_copy(acc.at[sl], rows.at[pl.ds(TA, RB)])
            if xdst is not None:
                pltpu.sync_copy(rows.at[pl.ds(TA, RB)], xdst.at[sl])
            pltpu.sync_copy(src_ref.at[sl], rows.at[pl.ds(TB, RB)])

            def urow(r, carry2):
                z16 = lax.broadcast(jnp.float32(0.0), (16,))
                a0 = rows[TA + r, 0:16]
                a1 = rows[TA + r, 16:32]
                b0 = rows[TB + r, 0:16]
                b1 = rows[TB + r, 16:32]
                if first:
                    rows[TB + r, 0:16] = (a0 + b0) * ALPHA
                    rows[TB + r, 16:32] = (a1 + b1) * ALPHA
                else:
                    rows[TB + r, 0:16] = b0 + a0 * ALPHA
                    rows[TB + r, 16:32] = b1 + a1 * ALPHA
                rows[TA + r, 0:16] = z16
                rows[TA + r, 16:32] = z16
                return carry2
            lax.fori_loop(0, RB, urow, 0, unroll=2)
            pltpu.sync_copy(rows.at[pl.ds(TB, RB)], out_ref.at[sl])
            pltpu.sync_copy(rows.at[pl.ds(TA, RB)], acc.at[sl])
            return carry
        lax.fori_loop(0, NRB, dblk, 0)

    ux0c, rx0c = ux0.at[cid], rx0.at[cid]
    uoutc, routc = uout.at[cid], rout.at[cid]
    xuc, xrc = xu.at[cid], xr.at[cid]

    convs = [
        (ux0c, se_ur, de_ur, w_ur, rx0c, routc, xrc, True),
        (xrc, se_ru, de_ru, w_ru, ux0c, uoutc, xuc, True),
        (xuc, se_ur, de_ur, w_ur, routc, routc, xrc, False),
        (xrc, se_ru, de_ru, w_ru, uoutc, uoutc, xuc, False),
        (xuc, se_ur, de_ur, w_ur, routc, routc, xrc, False),
        (xrc, se_ru, de_ru, w_ru, uoutc, uoutc, None, False),
    ]
    for xsrc, se, de, wh, src_ref, out_ref, xdst, first in convs:
        scatter_phase(xsrc, se, de, wh)
        plsc.subcore_barrier()
        drain(src_ref, out_ref, xdst, first)
        plsc.subcore_barrier()


@functools.cache
def _sc_kernel():
    return functools.partial(
        pl.kernel,
        out_type=[
            jax.ShapeDtypeStruct((NC, NP, C), jnp.float32),
            jax.ShapeDtypeStruct((NC, NP, C), jnp.float32),
            jax.ShapeDtypeStruct((NC, NP, C), jnp.float32),
            jax.ShapeDtypeStruct((NC, NP, C), jnp.float32),
        ],
        mesh=plsc.VectorSubcoreMesh(core_axis_name="c", subcore_axis_name="s",
                                    num_cores=NC, num_subcores=NS),
        compiler_params=pltpu.CompilerParams(use_tc_tiling_on_sc=False),
        scratch_types=[
            pltpu.VMEM_SHARED((NP, C), jnp.float32),
            pltpu.VMEM((KB * 128, C), jnp.float32),
            pltpu.VMEM((KB, 128), jnp.int32),
            pltpu.VMEM((KB, 128), jnp.int32),
            pltpu.VMEM((KB, 128), jnp.float32),
        ],
    )(_body)


def _prep_edges(edges, w):
    pad_idx = (jnp.arange(EPAD, dtype=jnp.int32) * 97) % N
    se = jnp.concatenate([edges[0], pad_idx]).reshape(EROWS, 128)
    de = jnp.concatenate([edges[1], pad_idx]).reshape(EROWS, 128)
    ww = jnp.concatenate([w, jnp.zeros((EPAD,), jnp.float32)]).reshape(EROWS, 128)
    return se, de, ww


def kernel(recipe_x, usr_rcp_edges, rcp_usr_edges, usr_rcp_weights,
           rcp_usr_weights, usr_emb, rcp_emb):
    rcp_x0 = jnp.concatenate([rcp_emb, recipe_x], axis=1)
    zpad = jnp.zeros((NP - N, D), jnp.float32)
    usr_p = jnp.concatenate([usr_emb, zpad], axis=0)
    rcp_p = jnp.concatenate([rcp_x0, zpad], axis=0)
    ux0 = jnp.stack([usr_p[:, :C], usr_p[:, C:]])
    rx0 = jnp.stack([rcp_p[:, :C], rcp_p[:, C:]])
    se_ur, de_ur, w_ur = _prep_edges(usr_rcp_edges, usr_rcp_weights)
    se_ru, de_ru, w_ru = _prep_edges(rcp_usr_edges, rcp_usr_weights)
    uout, rout, _, _ = _sc_kernel()(ux0, rx0, se_ur, de_ur, w_ur,
                                    se_ru, de_ru, w_ru)
    usr_out = jnp.concatenate([uout[0, :N], uout[1, :N]], axis=1)
    rec_out = jnp.concatenate([rout[0, :N], rout[1, :N]], axis=1)
    return (usr_out, rec_out)
